# baseline (speedup 1.0000x reference)
import functools

import jax
import jax.numpy as jnp
from jax import lax
from jax.experimental import pallas as pl
from jax.experimental.pallas import tpu as pltpu
from jax.experimental.pallas import tpu_sc as plsc

B = 16384
DIMS = (10, 10, 8, 8, 6, 6, 6, 6, 2, 2)
ROWS = (18, 19, 10, 11, 14, 6, 3, 9, 2, 2)
NF = len(DIMS)
D_OUT = sum(DIMS)

COL_OFF = []
_acc = 0
for _d in DIMS:
    COL_OFF.append(_acc)
    _acc += _d

FIELD_OF_COL = []
for _f, _d in enumerate(DIMS):
    FIELD_OF_COL.extend([_f] * _d)

TRANS = (True, True, True, True, True, False, False, True, False, False)

NC, NS, L = 2, 16, 16
NW = NC * NS
CHUNK = B // NW
UNROLL = 2

_mesh = plsc.VectorSubcoreMesh(core_axis_name="c", subcore_axis_name="s")


@functools.partial(
    pl.kernel,
    out_type=jax.ShapeDtypeStruct((D_OUT, B), jnp.float32),
    mesh=_mesh,
    compiler_params=pltpu.CompilerParams(
        needs_layout_passes=False, use_tc_tiling_on_sc=True),
    scratch_types=[
        pltpu.VMEM((NF, CHUNK), jnp.int32),
        pltpu.VMEM((D_OUT, CHUNK), jnp.float32),
        [pltpu.VMEM((d, n) if t else (n, d), jnp.float32)
         for n, d, t in zip(ROWS, DIMS, TRANS)],
        pltpu.SemaphoreType.DMA,
    ],
)
def _embed_sc(i0, i1, i2, i3, i4, i5, i6, i7, i8, i9,
              t0, t1, t2, t3, t4, t5, t6, t7, t8, t9,
              out_hbm, stage_v, out_v, tbl_vs, sem):
    wid = lax.axis_index("s") * NC + lax.axis_index("c")
    base = wid * CHUNK

    with jax.named_scope("stage_in"):
        idx_refs = (i0, i1, i2, i3, i4, i5, i6, i7, i8, i9)
        tbl_refs = (t0, t1, t2, t3, t4, t5, t6, t7, t8, t9)
        copies = [pltpu.make_async_copy(tbl_refs[f], tbl_vs[f], sem)
                  for f in range(NF)]
        copies += [
            pltpu.make_async_copy(
                idx_refs[f].at[pl.ds(base, CHUNK)], stage_v.at[f], sem)
            for f in range(NF)
        ]
        for c in copies:
            c.start()
        for c in copies:
            c.wait()

    with jax.named_scope("col_loop"):
        @plsc.parallel_loop(0, CHUNK // L, unroll=UNROLL)
        def _grp_loop(g):
            r0 = g * L
            raws = [stage_v[f, pl.ds(r0, L)] for f in range(NF)]
            for c in range(D_OUT):
                f = FIELD_OF_COL[c]
                j = c - COL_OFF[f]
                jsplat = jnp.broadcast_to(jnp.int32(j), (L,))
                if TRANS[f]:
                    vals = plsc.load_gather(tbl_vs[f], [jsplat, raws[f]])
                else:
                    vals = plsc.load_gather(tbl_vs[f], [raws[f], jsplat])
                out_v[c, pl.ds(r0, L)] = vals

    with jax.named_scope("write_out"):
        pltpu.sync_copy(out_v, out_hbm.at[:, pl.ds(base, CHUNK)])


def kernel(type1, type2, primary_color, secondary_color, shape, size,
           evolution_stage, habitat, legendary, mythical,
           type1_table, type2_table, primary_color_table,
           secondary_color_table, shape_table, size_table,
           evolution_stage_table, habitat_table, legendary_table,
           mythical_table):
    idxs = [x.astype(jnp.int32) for x in
            (type1, type2, primary_color, secondary_color, shape, size,
             evolution_stage, habitat, legendary, mythical)]
    tables = (type1_table, type2_table, primary_color_table,
              secondary_color_table, shape_table, size_table,
              evolution_stage_table, habitat_table, legendary_table,
              mythical_table)
    out_t = _embed_sc(*idxs, *[t.T if tr else t
                               for t, tr in zip(tables, TRANS)])
    return out_t.T


# speedup vs baseline: 1.1603x; 1.1603x over previous
import functools

import jax
import jax.numpy as jnp
from jax import lax
from jax.experimental import pallas as pl
from jax.experimental.pallas import tpu as pltpu
from jax.experimental.pallas import tpu_sc as plsc

B = 16384
DIMS = (10, 10, 8, 8, 6, 6, 6, 6, 2, 2)
ROWS = (18, 19, 10, 11, 14, 6, 3, 9, 2, 2)
NF = len(DIMS)
D_OUT = sum(DIMS)

COL_OFF = []
_acc = 0
for _d in DIMS:
    COL_OFF.append(_acc)
    _acc += _d

FIELD_OF_COL = []
for _f, _d in enumerate(DIMS):
    FIELD_OF_COL.extend([_f] * _d)

TRANS = (True, True, True, True, True, False, False, True, False, False)

NC, NS, L = 2, 16, 16
NW = NC * NS
CHUNK = B // NW
UNROLL = 2

_mesh = plsc.VectorSubcoreMesh(core_axis_name="c", subcore_axis_name="s")


@functools.partial(
    pl.kernel,
    out_type=jax.ShapeDtypeStruct((D_OUT, B), jnp.float32),
    mesh=_mesh,
    compiler_params=pltpu.CompilerParams(
        needs_layout_passes=False, use_tc_tiling_on_sc=True),
    scratch_types=[
        pltpu.VMEM((NF, CHUNK), jnp.int32),
        pltpu.VMEM((D_OUT, CHUNK), jnp.float32),
        [pltpu.VMEM((d, n) if t else (n, d), jnp.float32)
         for n, d, t in zip(ROWS, DIMS, TRANS)],
        [pltpu.VMEM((DIMS[f], L), jnp.float32)
         for f in range(NF) if not TRANS[f]],
        pltpu.SemaphoreType.DMA,
    ],
)
def _embed_sc(i0, i1, i2, i3, i4, i5, i6, i7, i8, i9,
              t0, t1, t2, t3, t4, t5, t6, t7, t8, t9,
              out_hbm, stage_v, out_v, tbl_vs, tr_vs, sem):
    wid = lax.axis_index("s") * NC + lax.axis_index("c")
    base = wid * CHUNK

    with jax.named_scope("stage_in"):
        idx_refs = (i0, i1, i2, i3, i4, i5, i6, i7, i8, i9)
        tbl_refs = (t0, t1, t2, t3, t4, t5, t6, t7, t8, t9)
        copies = [pltpu.make_async_copy(tbl_refs[f], tbl_vs[f], sem)
                  for f in range(NF)]
        copies += [
            pltpu.make_async_copy(
                idx_refs[f].at[pl.ds(base, CHUNK)], stage_v.at[f], sem)
            for f in range(NF)
        ]
        for c in copies:
            c.start()
        for c in copies:
            c.wait()

    with jax.named_scope("transpose_small"):
        iota = lax.iota(jnp.int32, L)
        k = 0
        tcol = []
        for f in range(NF):
            if TRANS[f]:
                tcol.append(tbl_vs[f])
                continue
            mask = iota < ROWS[f]
            for j in range(DIMS[f]):
                jsplat = jnp.broadcast_to(jnp.int32(j), (L,))
                col = plsc.load_gather(tbl_vs[f], [iota, jsplat],
                                       mask=mask)
                tr_vs[k][j, pl.ds(0, L)] = col
            tcol.append(tr_vs[k])
            k += 1

    with jax.named_scope("col_loop"):
        @plsc.parallel_loop(0, CHUNK // L, unroll=UNROLL)
        def _grp_loop(g):
            r0 = g * L
            raws = [stage_v[f, pl.ds(r0, L)] for f in range(NF)]
            for c in range(D_OUT):
                f = FIELD_OF_COL[c]
                j = c - COL_OFF[f]
                jsplat = jnp.broadcast_to(jnp.int32(j), (L,))
                vals = plsc.load_gather(tcol[f], [jsplat, raws[f]])
                out_v[c, pl.ds(r0, L)] = vals

    with jax.named_scope("write_out"):
        pltpu.sync_copy(out_v, out_hbm.at[:, pl.ds(base, CHUNK)])


def kernel(type1, type2, primary_color, secondary_color, shape, size,
           evolution_stage, habitat, legendary, mythical,
           type1_table, type2_table, primary_color_table,
           secondary_color_table, shape_table, size_table,
           evolution_stage_table, habitat_table, legendary_table,
           mythical_table):
    idxs = [x.astype(jnp.int32) for x in
            (type1, type2, primary_color, secondary_color, shape, size,
             evolution_stage, habitat, legendary, mythical)]
    tables = (type1_table, type2_table, primary_color_table,
              secondary_color_table, shape_table, size_table,
              evolution_stage_table, habitat_table, legendary_table,
              mythical_table)
    out_t = _embed_sc(*idxs, *[t.T if tr else t
                               for t, tr in zip(tables, TRANS)])
    return out_t.T
